# Initial kernel scaffold; baseline (speedup 1.0000x reference)
import jax
import jax.numpy as jnp
from jax.experimental import pallas as pl


def kernel(x, edge_index, W):
    raise NotImplementedError("write your pallas kernel here")


# speedup vs baseline: 6.1161x; 6.1161x over previous
import jax
import jax.numpy as jnp
from jax import lax
from jax.experimental import pallas as pl
from jax.experimental.pallas import tpu as pltpu
from jax.experimental.pallas import tpu_sc as plsc

N_NODES = 10000
N_EDGES = 320000
D = 128

NC = 2
NS = 16
CH = 80
EDGES_PER_TILE = N_EDGES // (NC * NS)
NITER = EDGES_PER_TILE // CH
N_PAD = 10240
ROWS_PER_TILE = N_PAD // NS
ZROWS = 128


def _sc_body(x_hbm, src_hbm, dst_hbm, agg_out, cnt_out,
             src_idx, dst_idx, rows, ones_v, zbuf, cbuf,
             agg_sh, cnt_sh, sem):
    c = lax.axis_index("c")
    s = lax.axis_index("s")
    tile = c * NS + s

    zvec = jnp.zeros((16,), jnp.float32)
    onev = jnp.ones((16,), jnp.float32)

    def zrow(i, carry):
        for j in range(D // 16):
            zbuf[i, pl.ds(j * 16, 16)] = zvec
        return carry
    lax.fori_loop(0, ZROWS, zrow, 0)

    def crow(i, carry):
        cbuf[pl.ds(i * 16, 16)] = zvec
        return carry
    lax.fori_loop(0, ROWS_PER_TILE // 16, crow, 0)

    for j in range(CH // 16):
        ones_v[pl.ds(j * 16, 16)] = onev

    for r in range(ROWS_PER_TILE // ZROWS):
        row0 = s * ROWS_PER_TILE + r * ZROWS
        pltpu.sync_copy(zbuf, agg_sh.at[pl.ds(row0, ZROWS), :])
    pltpu.sync_copy(cbuf, cnt_sh.at[pl.ds(s * ROWS_PER_TILE, ROWS_PER_TILE)])
    plsc.subcore_barrier()

    def body(i, carry):
        base = tile * EDGES_PER_TILE + i * CH
        pltpu.sync_copy(src_hbm.at[pl.ds(base, CH)], src_idx)
        pltpu.sync_copy(dst_hbm.at[pl.ds(base, CH)], dst_idx)
        pltpu.async_copy(x_hbm.at[src_idx], rows, sem).wait()
        pltpu.sync_copy(rows, agg_sh.at[dst_idx], add=True)
        pltpu.sync_copy(ones_v, cnt_sh.at[dst_idx], add=True)
        return carry
    lax.fori_loop(0, NITER, body, 0)

    plsc.subcore_barrier()

    for r in range(ROWS_PER_TILE // ZROWS):
        row0 = s * ROWS_PER_TILE + r * ZROWS
        pltpu.sync_copy(agg_sh.at[pl.ds(row0, ZROWS), :], zbuf)
        pltpu.sync_copy(zbuf, agg_out.at[c, pl.ds(row0, ZROWS), :])
    pltpu.sync_copy(cnt_sh.at[pl.ds(s * ROWS_PER_TILE, ROWS_PER_TILE)], cbuf)
    pltpu.sync_copy(cbuf, cnt_out.at[c, pl.ds(s * ROWS_PER_TILE, ROWS_PER_TILE)])


def _combine_body(a_ref, c_ref, x_ref, wt_ref, o_ref):
    agg = a_ref[0] + a_ref[1]
    cnt = c_ref[:, 0:1] + c_ref[:, 1:2]
    inv = 1.0 / jnp.maximum(cnt, 1.0)
    xa = agg * inv
    h = jnp.dot(xa, wt_ref[...], preferred_element_type=jnp.float32)
    o_ref[...] = jnp.maximum(h, 0.0) + x_ref[...]


def kernel(x, edge_index, W):
    src = edge_index[0]
    dst = edge_index[1]

    mesh = plsc.VectorSubcoreMesh(core_axis_name="c", subcore_axis_name="s")
    sc_fn = pl.kernel(
        _sc_body,
        mesh=mesh,
        out_type=[
            jax.ShapeDtypeStruct((NC, N_PAD, D), jnp.float32),
            jax.ShapeDtypeStruct((NC, N_PAD), jnp.float32),
        ],
        scratch_types=[
            pltpu.VMEM((CH,), jnp.int32),
            pltpu.VMEM((CH,), jnp.int32),
            pltpu.VMEM((CH, D), jnp.float32),
            pltpu.VMEM((CH,), jnp.float32),
            pltpu.VMEM((ZROWS, D), jnp.float32),
            pltpu.VMEM((ROWS_PER_TILE,), jnp.float32),
            pltpu.VMEM_SHARED((N_PAD, D), jnp.float32),
            pltpu.VMEM_SHARED((N_PAD,), jnp.float32),
            pltpu.SemaphoreType.DMA,
        ],
    )
    agg_parts, cnt_parts = sc_fn(x, src, dst)
    cnt_t = cnt_parts.T

    blk = 512
    h_pad = pl.pallas_call(
        _combine_body,
        grid=(N_PAD // blk,),
        in_specs=[
            pl.BlockSpec((NC, blk, D), lambda i: (0, i, 0)),
            pl.BlockSpec((blk, NC), lambda i: (i, 0)),
            pl.BlockSpec((blk, D), lambda i: (i, 0)),
            pl.BlockSpec((D, D), lambda i: (0, 0)),
        ],
        out_specs=pl.BlockSpec((blk, D), lambda i: (i, 0)),
        out_shape=jax.ShapeDtypeStruct((N_PAD, D), jnp.float32),
    )(agg_parts, cnt_t, jnp.pad(x, ((0, N_PAD - N_NODES), (0, 0))), W.T)
    return h_pad[:N_NODES]
